# Initial kernel scaffold; baseline (speedup 1.0000x reference)
import jax
import jax.numpy as jnp
from jax.experimental import pallas as pl


def kernel(target, context, target_table, context_table):
    raise NotImplementedError("write your pallas kernel here")


# speedup vs baseline: 4.1911x; 4.1911x over previous
import functools

import jax
import jax.numpy as jnp
from jax import lax
from jax.experimental import pallas as pl
from jax.experimental.pallas import tpu as pltpu
from jax.experimental.pallas import tpu_sc as plsc

E = 128
C = 6
L = 16
NUM_CORES = 2
NUM_SUBCORES = 16
NW = NUM_CORES * NUM_SUBCORES
CB = 64
IDX_CHUNK = 128


def _build_sc_call(B):
    b_per_w = B // NW
    n_chunks = b_per_w // CB
    n_ctx_dma = (CB * C) // IDX_CHUNK

    mesh = plsc.VectorSubcoreMesh(
        core_axis_name="c", subcore_axis_name="s",
        num_cores=NUM_CORES, num_subcores=NUM_SUBCORES)

    @functools.partial(
        pl.kernel,
        out_type=jax.ShapeDtypeStruct((B * C,), jnp.float32),
        mesh=mesh,
        scratch_types=[
            pltpu.VMEM((CB,), jnp.int32),
            pltpu.VMEM((CB * C,), jnp.int32),
            pltpu.VMEM((CB, E), jnp.float32),
            pltpu.VMEM((CB * C, E), jnp.float32),
            pltpu.VMEM((CB * C,), jnp.float32),
            pltpu.SemaphoreType.DMA,
        ],
    )
    def sc_call(tgt_hbm, ctx_hbm, ttab_hbm, ctab_hbm, out_hbm,
                tidx, cidx, wrows, crows, outv, sem):
        wid = lax.axis_index("s") * NUM_CORES + lax.axis_index("c")
        base = wid * b_per_w
        for ch in range(n_chunks):
            b0 = base + ch * CB
            pltpu.sync_copy(tgt_hbm.at[pl.ds(b0, CB)], tidx)
            pltpu.sync_copy(ctx_hbm.at[pl.ds(b0 * C, CB * C)], cidx)

            copies = [pltpu.async_copy(ttab_hbm.at[tidx], wrows, sem)]
            for k in range(n_ctx_dma):
                copies.append(pltpu.async_copy(
                    ctab_hbm.at[cidx.at[pl.ds(k * IDX_CHUNK, IDX_CHUNK)]],
                    crows.at[pl.ds(k * IDX_CHUNK, IDX_CHUNK)], sem))
            for cp in copies:
                cp.wait()

            BG = 8
            NV = (BG * C) // L
            lane = lax.iota(jnp.int32, L)
            rots = [((lane + sh) & (L - 1)).reshape(L, 1) for sh in (8, 4, 2, 1)]
            dnums = lax.GatherDimensionNumbers(
                offset_dims=(), collapsed_slice_dims=(0,), start_index_map=(0,))

            def lane_sum(v):
                for idx in rots:
                    v = v + lax.gather(
                        v, idx, dnums, slice_sizes=(1,),
                        mode=lax.GatherScatterMode.PROMISE_IN_BOUNDS)
                return v

            def grp_body(g, carry):
                b0g = g * BG
                res = [jnp.zeros((L,), jnp.float32) for _ in range(NV)]
                for bb in range(BG):
                    b = b0g + bb
                    wv = [wrows[b, pl.ds(L * j, L)] for j in range(E // L)]
                    for c in range(C):
                        row = b * C + c
                        acc = wv[0] * crows[row, pl.ds(0, L)]
                        for j in range(1, E // L):
                            acc = acc + wv[j] * crows[row, pl.ds(L * j, L)]
                        s = lane_sum(acc)
                        v, p = divmod(bb * C + c, L)
                        res[v] = jnp.where(lane == p, s, res[v])
                for v in range(NV):
                    outv[pl.ds(g * BG * C + v * L, L)] = res[v]
                return carry

            lax.fori_loop(0, CB // BG, grp_body, 0)
            pltpu.sync_copy(outv, out_hbm.at[pl.ds(b0 * C, CB * C)])

    return sc_call


def kernel(target, context, target_table, context_table):
    if target.ndim == 2:
        target = jnp.squeeze(target, axis=1)
    B = target.shape[0]
    tgt = target.astype(jnp.int32)
    ctx = context.astype(jnp.int32).reshape(-1)
    out = _build_sc_call(B)(tgt, ctx, target_table, context_table)
    return out.reshape(B, C)


# speedup vs baseline: 5.1071x; 1.2186x over previous
import functools

import jax
import jax.numpy as jnp
from jax import lax
from jax.experimental import pallas as pl
from jax.experimental.pallas import tpu as pltpu
from jax.experimental.pallas import tpu_sc as plsc

E = 128
C = 6
L = 16
NUM_CORES = 2
NUM_SUBCORES = 16
NW = NUM_CORES * NUM_SUBCORES
CB = 64
IDX_CHUNK = 128
BG = 8


def _build_sc_call(B):
    b_per_w = B // NW
    n_chunks = b_per_w // CB
    n_ctx_dma = (CB * C) // IDX_CHUNK

    mesh = plsc.VectorSubcoreMesh(
        core_axis_name="c", subcore_axis_name="s",
        num_cores=NUM_CORES, num_subcores=NUM_SUBCORES)

    buf_types = [
        pltpu.VMEM((CB,), jnp.int32),
        pltpu.VMEM((CB * C,), jnp.int32),
        pltpu.VMEM((CB, E), jnp.float32),
        pltpu.VMEM((CB * C, E), jnp.float32),
        pltpu.SemaphoreType.DMA,
    ]

    @functools.partial(
        pl.kernel,
        out_type=jax.ShapeDtypeStruct((B * C,), jnp.float32),
        mesh=mesh,
        scratch_types=buf_types + buf_types + [
            pltpu.VMEM((CB * C,), jnp.float32),
        ],
    )
    def sc_call(tgt_hbm, ctx_hbm, ttab_hbm, ctab_hbm, out_hbm,
                tidx0, cidx0, wrows0, crows0, sem0,
                tidx1, cidx1, wrows1, crows1, sem1,
                outv):
        wid = lax.axis_index("s") * NUM_CORES + lax.axis_index("c")
        base = wid * b_per_w
        bufs = [(tidx0, cidx0, wrows0, crows0, sem0),
                (tidx1, cidx1, wrows1, crows1, sem1)]

        def issue(ch, buf):
            tidx, cidx, wrows, crows, sem = buf
            b0 = base + ch * CB
            pltpu.sync_copy(tgt_hbm.at[pl.ds(b0, CB)], tidx)
            pltpu.sync_copy(ctx_hbm.at[pl.ds(b0 * C, CB * C)], cidx)
            cps = [pltpu.async_copy(ttab_hbm.at[tidx], wrows, sem)]
            for k in range(n_ctx_dma):
                cps.append(pltpu.async_copy(
                    ctab_hbm.at[cidx.at[pl.ds(k * IDX_CHUNK, IDX_CHUNK)]],
                    crows.at[pl.ds(k * IDX_CHUNK, IDX_CHUNK)], sem))
            return cps

        NV = (BG * C) // L
        lane = lax.iota(jnp.int32, L)
        rots = [((lane + sh) & (L - 1)).reshape(L, 1) for sh in (8, 4, 2, 1)]
        dnums = lax.GatherDimensionNumbers(
            offset_dims=(), collapsed_slice_dims=(0,), start_index_map=(0,))

        def lane_sum(v):
            for idx in rots:
                v = v + lax.gather(
                    v, idx, dnums, slice_sizes=(1,),
                    mode=lax.GatherScatterMode.PROMISE_IN_BOUNDS)
            return v

        def compute(ch, buf):
            _, _, wrows, crows, _ = buf
            b0 = base + ch * CB

            def grp_body(g, carry):
                b0g = g * BG
                res = [jnp.zeros((L,), jnp.float32) for _ in range(NV)]
                for bb in range(BG):
                    b = b0g + bb
                    wv = [wrows[b, pl.ds(L * j, L)] for j in range(E // L)]
                    for c in range(C):
                        row = b * C + c
                        acc = wv[0] * crows[row, pl.ds(0, L)]
                        for j in range(1, E // L):
                            acc = acc + wv[j] * crows[row, pl.ds(L * j, L)]
                        s = lane_sum(acc)
                        v, p = divmod(bb * C + c, L)
                        res[v] = jnp.where(lane == p, s, res[v])
                for v in range(NV):
                    outv[pl.ds(g * BG * C + v * L, L)] = res[v]
                return carry

            lax.fori_loop(0, CB // BG, grp_body, 0)
            pltpu.sync_copy(outv, out_hbm.at[pl.ds(b0 * C, CB * C)])

        pending = issue(0, bufs[0])
        for ch in range(n_chunks):
            for cp in pending:
                cp.wait()
            if ch + 1 < n_chunks:
                pending = issue(ch + 1, bufs[(ch + 1) % 2])
            compute(ch, bufs[ch % 2])

    return sc_call


def kernel(target, context, target_table, context_table):
    if target.ndim == 2:
        target = jnp.squeeze(target, axis=1)
    B = target.shape[0]
    tgt = target.astype(jnp.int32)
    ctx = context.astype(jnp.int32).reshape(-1)
    out = _build_sc_call(B)(tgt, ctx, target_table, context_table)
    return out.reshape(B, C)


# speedup vs baseline: 5.6225x; 1.1009x over previous
import functools

import jax
import jax.numpy as jnp
from jax import lax
from jax.experimental import pallas as pl
from jax.experimental.pallas import tpu as pltpu
from jax.experimental.pallas import tpu_sc as plsc

E = 128
C = 6
L = 16
NUM_CORES = 2
NUM_SUBCORES = 16
NW = NUM_CORES * NUM_SUBCORES
CB = 64
IDX_CHUNK = 128
BG = 8


def _build_sc_call(B):
    b_per_w = B // NW
    n_chunks = b_per_w // CB
    n_ctx_dma = (CB * C) // IDX_CHUNK

    mesh = plsc.VectorSubcoreMesh(
        core_axis_name="c", subcore_axis_name="s",
        num_cores=NUM_CORES, num_subcores=NUM_SUBCORES)

    buf_types = [
        pltpu.VMEM((CB,), jnp.int32),
        pltpu.VMEM((CB * C,), jnp.int32),
        pltpu.VMEM((CB, E), jnp.float32),
        pltpu.VMEM((CB * C, E), jnp.float32),
        pltpu.SemaphoreType.DMA,
    ]

    @functools.partial(
        pl.kernel,
        out_type=jax.ShapeDtypeStruct((B * C,), jnp.float32),
        mesh=mesh,
        scratch_types=buf_types + buf_types + [
            pltpu.VMEM((CB * C,), jnp.float32),
        ],
    )
    def sc_call(tgt_hbm, ctx_hbm, ttab_hbm, ctab_hbm, out_hbm,
                tidx0, cidx0, wrows0, crows0, sem0,
                tidx1, cidx1, wrows1, crows1, sem1,
                outv):
        wid = lax.axis_index("s") * NUM_CORES + lax.axis_index("c")
        base = wid * b_per_w
        bufs = [(tidx0, cidx0, wrows0, crows0, sem0),
                (tidx1, cidx1, wrows1, crows1, sem1)]

        def issue(ch, buf):
            tidx, cidx, wrows, crows, sem = buf
            b0 = base + ch * CB
            pltpu.sync_copy(tgt_hbm.at[pl.ds(b0, CB)], tidx)
            pltpu.sync_copy(ctx_hbm.at[pl.ds(b0 * C, CB * C)], cidx)
            cps = [pltpu.async_copy(ttab_hbm.at[tidx], wrows, sem)]
            for k in range(n_ctx_dma):
                cps.append(pltpu.async_copy(
                    ctab_hbm.at[cidx.at[pl.ds(k * IDX_CHUNK, IDX_CHUNK)]],
                    crows.at[pl.ds(k * IDX_CHUNK, IDX_CHUNK)], sem))
            return cps

        NV = (BG * C) // L
        lane = lax.iota(jnp.int32, L)
        xors = {sh: (lane ^ sh).reshape(L, 1) for sh in (8, 4, 2, 1)}
        dnums = lax.GatherDimensionNumbers(
            offset_dims=(), collapsed_slice_dims=(0,), start_index_map=(0,))

        def swap(v, sh):
            return lax.gather(
                v, xors[sh], dnums, slice_sizes=(1,),
                mode=lax.GatherScatterMode.PROMISE_IN_BOUNDS)

        def merge_tree(grp):
            sh = L // 2
            while len(grp) > 1:
                half = len(grp) // 2
                nxt = []
                for i in range(half):
                    u = grp[i] + swap(grp[i], sh)
                    w = grp[i + half] + swap(grp[i + half], sh)
                    nxt.append(jnp.where((lane & sh) == 0, u, w))
                grp = nxt
                sh //= 2
            return grp[0]

        def compute(ch, buf):
            _, _, wrows, crows, _ = buf
            b0 = base + ch * CB

            def grp_body(g, carry):
                b0g = g * BG
                wv_cache = {}
                grp = []
                v = 0
                for rl in range(BG * C):
                    bb, c = divmod(rl, C)
                    if bb not in wv_cache:
                        wv_cache = {bb: [wrows[b0g + bb, pl.ds(L * j, L)]
                                         for j in range(E // L)]}
                    wv = wv_cache[bb]
                    row = (b0g + bb) * C + c
                    ps = [wv[j] * crows[row, pl.ds(L * j, L)]
                          for j in range(E // L)]
                    while len(ps) > 1:
                        ps = [ps[i] + ps[i + 1] for i in range(0, len(ps), 2)]
                    grp.append(ps[0])
                    if len(grp) == L:
                        outv[pl.ds(g * BG * C + v * L, L)] = merge_tree(grp)
                        grp = []
                        v += 1
                return carry

            lax.fori_loop(0, CB // BG, grp_body, 0)
            pltpu.sync_copy(outv, out_hbm.at[pl.ds(b0 * C, CB * C)])

        pending = issue(0, bufs[0])
        for ch in range(n_chunks):
            for cp in pending:
                cp.wait()
            if ch + 1 < n_chunks:
                pending = issue(ch + 1, bufs[(ch + 1) % 2])
            compute(ch, bufs[ch % 2])

    return sc_call


def kernel(target, context, target_table, context_table):
    if target.ndim == 2:
        target = jnp.squeeze(target, axis=1)
    B = target.shape[0]
    tgt = target.astype(jnp.int32)
    ctx = context.astype(jnp.int32).reshape(-1)
    out = _build_sc_call(B)(tgt, ctx, target_table, context_table)
    return out.reshape(B, C)


# speedup vs baseline: 5.9747x; 1.0626x over previous
import functools

import jax
import jax.numpy as jnp
from jax import lax
from jax.experimental import pallas as pl
from jax.experimental.pallas import tpu as pltpu
from jax.experimental.pallas import tpu_sc as plsc

E = 128
C = 6
L = 16
NUM_CORES = 2
NUM_SUBCORES = 16
NW = NUM_CORES * NUM_SUBCORES
CB = 64
IDX_CHUNK = 128
BG = 8


def _build_sc_call(B):
    b_per_w = B // NW
    n_chunks = b_per_w // CB
    n_ctx_dma = (CB * C) // IDX_CHUNK

    mesh = plsc.VectorSubcoreMesh(
        core_axis_name="c", subcore_axis_name="s",
        num_cores=NUM_CORES, num_subcores=NUM_SUBCORES)

    buf_types = [
        pltpu.VMEM((CB, E), jnp.float32),
        pltpu.VMEM((CB * C, E), jnp.float32),
        pltpu.VMEM((CB * C,), jnp.float32),
        pltpu.SemaphoreType.DMA,
        pltpu.SemaphoreType.DMA,
    ]

    @functools.partial(
        pl.kernel,
        out_type=jax.ShapeDtypeStruct((B * C,), jnp.float32),
        mesh=mesh,
        scratch_types=buf_types + buf_types + [
            pltpu.VMEM((b_per_w,), jnp.int32),
            pltpu.VMEM((b_per_w * C,), jnp.int32),
        ],
    )
    def sc_call(tgt_hbm, ctx_hbm, ttab_hbm, ctab_hbm, out_hbm,
                wrows0, crows0, outv0, sem0, semo0,
                wrows1, crows1, outv1, sem1, semo1,
                tidx, cidx):
        wid = lax.axis_index("s") * NUM_CORES + lax.axis_index("c")
        base = wid * b_per_w
        bufs = [(wrows0, crows0, outv0, sem0, semo0),
                (wrows1, crows1, outv1, sem1, semo1)]

        pltpu.sync_copy(tgt_hbm.at[pl.ds(base, b_per_w)], tidx)
        pltpu.sync_copy(ctx_hbm.at[pl.ds(base * C, b_per_w * C)], cidx)

        def issue(ch, buf):
            wrows, crows, _, sem, _ = buf
            cps = [pltpu.async_copy(
                ttab_hbm.at[tidx.at[pl.ds(ch * CB, CB)]], wrows, sem)]
            for k in range(n_ctx_dma):
                cps.append(pltpu.async_copy(
                    ctab_hbm.at[cidx.at[pl.ds(ch * CB * C + k * IDX_CHUNK,
                                              IDX_CHUNK)]],
                    crows.at[pl.ds(k * IDX_CHUNK, IDX_CHUNK)], sem))
            return cps

        NV = (BG * C) // L
        lane = lax.iota(jnp.int32, L)
        xors = {sh: (lane ^ sh).reshape(L, 1) for sh in (8, 4, 2, 1)}
        dnums = lax.GatherDimensionNumbers(
            offset_dims=(), collapsed_slice_dims=(0,), start_index_map=(0,))

        def swap(v, sh):
            return lax.gather(
                v, xors[sh], dnums, slice_sizes=(1,),
                mode=lax.GatherScatterMode.PROMISE_IN_BOUNDS)

        def merge_tree(grp):
            sh = L // 2
            while len(grp) > 1:
                half = len(grp) // 2
                nxt = []
                for i in range(half):
                    u = grp[i] + swap(grp[i], sh)
                    w = grp[i + half] + swap(grp[i + half], sh)
                    nxt.append(jnp.where((lane & sh) == 0, u, w))
                grp = nxt
                sh //= 2
            return grp[0]

        def compute(ch, buf):
            wrows, crows, outv, _, _ = buf

            def grp_body(g, carry):
                b0g = g * BG
                wv_cache = {}
                grp = []
                v = 0
                for rl in range(BG * C):
                    bb, c = divmod(rl, C)
                    if bb not in wv_cache:
                        wv_cache = {bb: [wrows[b0g + bb, pl.ds(L * j, L)]
                                         for j in range(E // L)]}
                    wv = wv_cache[bb]
                    row = (b0g + bb) * C + c
                    ps = [wv[j] * crows[row, pl.ds(L * j, L)]
                          for j in range(E // L)]
                    while len(ps) > 1:
                        ps = [ps[i] + ps[i + 1] for i in range(0, len(ps), 2)]
                    grp.append(ps[0])
                    if len(grp) == L:
                        outv[pl.ds(g * BG * C + v * L, L)] = merge_tree(grp)
                        grp = []
                        v += 1
                return carry

            lax.fori_loop(0, CB // BG, grp_body, 0)

        pend_gather = {0: issue(0, bufs[0])}
        if n_chunks > 1:
            pend_gather[1] = issue(1, bufs[1])
        pend_out = {}
        for ch in range(n_chunks):
            buf = bufs[ch % 2]
            for cp in pend_gather.pop(ch):
                cp.wait()
            if ch - 2 in pend_out:
                pend_out.pop(ch - 2).wait()
            compute(ch, buf)
            b0 = base + ch * CB
            pend_out[ch] = pltpu.async_copy(
                buf[2], out_hbm.at[pl.ds(b0 * C, CB * C)], buf[4])
            if ch + 2 < n_chunks:
                pend_gather[ch + 2] = issue(ch + 2, buf)
        for cp in pend_out.values():
            cp.wait()

    return sc_call


def kernel(target, context, target_table, context_table):
    if target.ndim == 2:
        target = jnp.squeeze(target, axis=1)
    B = target.shape[0]
    tgt = target.astype(jnp.int32)
    ctx = context.astype(jnp.int32).reshape(-1)
    out = _build_sc_call(B)(tgt, ctx, target_table, context_table)
    return out.reshape(B, C)


# speedup vs baseline: 5.9748x; 1.0000x over previous
import functools

import jax
import jax.numpy as jnp
from jax import lax
from jax.experimental import pallas as pl
from jax.experimental.pallas import tpu as pltpu
from jax.experimental.pallas import tpu_sc as plsc

E = 128
C = 6
L = 16
NUM_CORES = 2
NUM_SUBCORES = 16
NW = NUM_CORES * NUM_SUBCORES
CB = 64
IDX_CHUNK = 128
BG = 8


def _build_sc_call(B):
    b_per_w = B // NW
    n_chunks = b_per_w // CB
    n_ctx_dma = (CB * C) // IDX_CHUNK

    mesh = plsc.VectorSubcoreMesh(
        core_axis_name="c", subcore_axis_name="s",
        num_cores=NUM_CORES, num_subcores=NUM_SUBCORES)

    buf_types = [
        pltpu.VMEM((CB, E), jnp.float32),
        pltpu.VMEM((CB * C, E), jnp.float32),
        pltpu.VMEM((CB * C,), jnp.float32),
        pltpu.SemaphoreType.DMA,
        pltpu.SemaphoreType.DMA,
    ]

    @functools.partial(
        pl.kernel,
        out_type=jax.ShapeDtypeStruct((B * C,), jnp.float32),
        mesh=mesh,
        scratch_types=buf_types + buf_types + [
            pltpu.VMEM((b_per_w,), jnp.int32),
            pltpu.VMEM((b_per_w * C,), jnp.int32),
        ],
    )
    def sc_call(tgt_hbm, ctx_hbm, ttab_hbm, ctab_hbm, out_hbm,
                wrows0, crows0, outv0, sem0, semo0,
                wrows1, crows1, outv1, sem1, semo1,
                tidx, cidx):
        wid = lax.axis_index("s") * NUM_CORES + lax.axis_index("c")
        base = wid * b_per_w
        bufs = [(wrows0, crows0, outv0, sem0, semo0),
                (wrows1, crows1, outv1, sem1, semo1)]

        head = min(2 * CB, b_per_w)
        pltpu.sync_copy(tgt_hbm.at[pl.ds(base, head)], tidx.at[pl.ds(0, head)])
        pltpu.sync_copy(ctx_hbm.at[pl.ds(base * C, head * C)],
                        cidx.at[pl.ds(0, head * C)])

        def issue(ch, buf):
            wrows, crows, _, sem, _ = buf
            cps = [pltpu.async_copy(
                ttab_hbm.at[tidx.at[pl.ds(ch * CB, CB)]], wrows, sem)]
            for k in range(n_ctx_dma):
                cps.append(pltpu.async_copy(
                    ctab_hbm.at[cidx.at[pl.ds(ch * CB * C + k * IDX_CHUNK,
                                              IDX_CHUNK)]],
                    crows.at[pl.ds(k * IDX_CHUNK, IDX_CHUNK)], sem))
            return cps

        NV = (BG * C) // L
        lane = lax.iota(jnp.int32, L)
        xors = {sh: (lane ^ sh).reshape(L, 1) for sh in (8, 4, 2, 1)}
        dnums = lax.GatherDimensionNumbers(
            offset_dims=(), collapsed_slice_dims=(0,), start_index_map=(0,))

        def swap(v, sh):
            return lax.gather(
                v, xors[sh], dnums, slice_sizes=(1,),
                mode=lax.GatherScatterMode.PROMISE_IN_BOUNDS)

        def merge_tree(grp):
            sh = L // 2
            while len(grp) > 1:
                half = len(grp) // 2
                nxt = []
                for i in range(half):
                    u = grp[i] + swap(grp[i], sh)
                    w = grp[i + half] + swap(grp[i + half], sh)
                    nxt.append(jnp.where((lane & sh) == 0, u, w))
                grp = nxt
                sh //= 2
            return grp[0]

        def compute(ch, buf):
            wrows, crows, outv, _, _ = buf

            def grp_body(g, carry):
                b0g = g * BG
                wv_cache = {}
                grp = []
                v = 0
                for rl in range(BG * C):
                    bb, c = divmod(rl, C)
                    if bb not in wv_cache:
                        wv_cache = {bb: [wrows[b0g + bb, pl.ds(L * j, L)]
                                         for j in range(E // L)]}
                    wv = wv_cache[bb]
                    row = (b0g + bb) * C + c
                    ps = [wv[j] * crows[row, pl.ds(L * j, L)]
                          for j in range(E // L)]
                    while len(ps) > 1:
                        ps = [ps[i] + ps[i + 1] for i in range(0, len(ps), 2)]
                    grp.append(ps[0])
                    if len(grp) == L:
                        outv[pl.ds(g * BG * C + v * L, L)] = merge_tree(grp)
                        grp = []
                        v += 1
                return carry

            lax.fori_loop(0, CB // BG, grp_body, 0)

        pend_gather = {0: issue(0, bufs[0])}
        if n_chunks > 1:
            pend_gather[1] = issue(1, bufs[1])
        if b_per_w > head:
            pltpu.sync_copy(tgt_hbm.at[pl.ds(base + head, b_per_w - head)],
                            tidx.at[pl.ds(head, b_per_w - head)])
            pltpu.sync_copy(ctx_hbm.at[pl.ds((base + head) * C,
                                             (b_per_w - head) * C)],
                            cidx.at[pl.ds(head * C, (b_per_w - head) * C)])
        pend_out = {}
        for ch in range(n_chunks):
            buf = bufs[ch % 2]
            for cp in pend_gather.pop(ch):
                cp.wait()
            if ch - 2 in pend_out:
                pend_out.pop(ch - 2).wait()
            compute(ch, buf)
            b0 = base + ch * CB
            pend_out[ch] = pltpu.async_copy(
                buf[2], out_hbm.at[pl.ds(b0 * C, CB * C)], buf[4])
            if ch + 2 < n_chunks:
                pend_gather[ch + 2] = issue(ch + 2, buf)
        for cp in pend_out.values():
            cp.wait()

    return sc_call


def kernel(target, context, target_table, context_table):
    if target.ndim == 2:
        target = jnp.squeeze(target, axis=1)
    B = target.shape[0]
    tgt = target.astype(jnp.int32)
    ctx = context.astype(jnp.int32).reshape(-1)
    out = _build_sc_call(B)(tgt, ctx, target_table, context_table)
    return out.reshape(B, C)


# speedup vs baseline: 6.8159x; 1.1408x over previous
import functools

import jax
import jax.numpy as jnp
from jax import lax
from jax.experimental import pallas as pl
from jax.experimental.pallas import tpu as pltpu
from jax.experimental.pallas import tpu_sc as plsc

E = 128
C = 6
L = 16
NUM_CORES = 2
NUM_SUBCORES = 16
NW = NUM_CORES * NUM_SUBCORES
CB = 32
OT = 128


def _build_sc_call(B):
    b_per_w = B // NW
    n_chunks = b_per_w // CB

    mesh = plsc.VectorSubcoreMesh(
        core_axis_name="c", subcore_axis_name="s",
        num_cores=NUM_CORES, num_subcores=NUM_SUBCORES)

    buf_types = [
        pltpu.VMEM((CB, E), jnp.float32),
        pltpu.VMEM((C * CB, E), jnp.float32),
        pltpu.SemaphoreType.DMA,
    ]
    out_buf_types = [
        pltpu.VMEM((C, OT), jnp.float32),
        pltpu.SemaphoreType.DMA,
    ]
    cpo = OT // CB

    @functools.partial(
        pl.kernel,
        out_type=jax.ShapeDtypeStruct((C, B), jnp.float32),
        mesh=mesh,
        scratch_types=buf_types + buf_types + out_buf_types + out_buf_types + [
            pltpu.VMEM((b_per_w,), jnp.int32),
            pltpu.VMEM((C, b_per_w), jnp.int32),
        ],
    )
    def sc_call(tgt_hbm, ctxT_hbm, ttab_hbm, ctab_hbm, outT_hbm,
                wrows0, crows0, sem0,
                wrows1, crows1, sem1,
                outvT0, semo0, outvT1, semo1,
                tidx, cidx):
        wid = lax.axis_index("s") * NUM_CORES + lax.axis_index("c")
        base = wid * b_per_w
        bufs = [(wrows0, crows0, sem0), (wrows1, crows1, sem1)]
        obufs = [(outvT0, semo0), (outvT1, semo1)]

        head = min(OT, b_per_w)
        pltpu.sync_copy(tgt_hbm.at[pl.ds(base, head)], tidx.at[pl.ds(0, head)])
        pltpu.sync_copy(ctxT_hbm.at[:, pl.ds(base, head)],
                        cidx.at[:, pl.ds(0, head)])

        def issue(ch, buf):
            wrows, crows, sem = buf
            cps = [pltpu.async_copy(
                ttab_hbm.at[tidx.at[pl.ds(ch * CB, CB)]], wrows, sem)]
            for c in range(C):
                cps.append(pltpu.async_copy(
                    ctab_hbm.at[cidx.at[c, pl.ds(ch * CB, CB)]],
                    crows.at[pl.ds(c * CB, CB)], sem))
            return cps

        lane = lax.iota(jnp.int32, L)
        xors = {sh: (lane ^ sh).reshape(L, 1) for sh in (8, 4, 2, 1)}
        dnums = lax.GatherDimensionNumbers(
            offset_dims=(), collapsed_slice_dims=(0,), start_index_map=(0,))

        def swap(v, sh):
            return lax.gather(
                v, xors[sh], dnums, slice_sizes=(1,),
                mode=lax.GatherScatterMode.PROMISE_IN_BOUNDS)

        def compute(ch, buf, outvT):
            wrows, crows, _ = buf
            half = (ch % cpo) * CB

            def grp_body(g, part):
                blk = g // (C * 2)
                rem = g % (C * 2)
                c = rem // 2
                h = rem % 2
                b0 = blk * L + h * (L // 2)
                st = []
                for bb in range(L // 2):
                    wv = [wrows[b0 + bb, pl.ds(L * j, L)]
                          for j in range(E // L)]
                    ps = [wv[j] * crows[c * CB + b0 + bb, pl.ds(L * j, L)]
                          for j in range(E // L)]
                    while len(ps) > 1:
                        ps = [ps[k] + ps[k + 1]
                              for k in range(0, len(ps), 2)]
                    p, lvl = ps[0], 0
                    while st and st[-1][0] == lvl:
                        _, a = st.pop()
                        sh = 1 << lvl
                        u = a + swap(a, sh)
                        w = p + swap(p, sh)
                        p = jnp.where((lane & sh) == 0, u, w)
                        lvl += 1
                    st.append((lvl, p))
                q = st[-1][1]
                u = part + swap(part, 8)
                w = q + swap(q, 8)
                full = jnp.where((lane & 8) == 0, u, w)

                @pl.when(h == 1)
                def _():
                    outvT[c, pl.ds(half + blk * L, L)] = full

                return q

            zero = jnp.zeros((L,), jnp.float32)
            lax.fori_loop(0, (CB // L) * C * 2, grp_body, zero)

        pend_gather = {0: issue(0, bufs[0])}
        if n_chunks > 1:
            pend_gather[1] = issue(1, bufs[1])
        if b_per_w > head:
            pltpu.sync_copy(tgt_hbm.at[pl.ds(base + head, b_per_w - head)],
                            tidx.at[pl.ds(head, b_per_w - head)])
            pltpu.sync_copy(ctxT_hbm.at[:, pl.ds(base + head, b_per_w - head)],
                            cidx.at[:, pl.ds(head, b_per_w - head)])
        pend_out = {}
        for ch in range(n_chunks):
            buf = bufs[ch % 2]
            tile = ch // cpo
            outvT, semo = obufs[tile % 2]
            for cp in pend_gather.pop(ch):
                cp.wait()
            if ch % cpo == 0 and tile - 2 in pend_out:
                pend_out.pop(tile - 2).wait()
            compute(ch, buf, outvT)
            if ch % cpo == cpo - 1:
                b0 = base + tile * OT
                pend_out[tile] = pltpu.async_copy(
                    outvT, outT_hbm.at[:, pl.ds(b0, OT)], semo)
            if ch + 2 < n_chunks:
                pend_gather[ch + 2] = issue(ch + 2, buf)
        for cp in pend_out.values():
            cp.wait()

    return sc_call


def kernel(target, context, target_table, context_table):
    if target.ndim == 2:
        target = jnp.squeeze(target, axis=1)
    B = target.shape[0]
    tgt = target.astype(jnp.int32)
    ctxT = context.astype(jnp.int32).T
    outT = _build_sc_call(B)(tgt, ctxT, target_table, context_table)
    return outT.T
